# baseline (speedup 1.0000x reference)
import functools

import jax
import jax.numpy as jnp
from jax import lax
from jax.experimental import pallas as pl
from jax.experimental.pallas import tpu as pltpu
from jax.experimental.pallas import tpu_sc as plsc

_N = 768
_D = 128
_K = 7
_F = _K + 1
_NW = 16
_NPW = _N // _NW
_NG = _NPW // 16


def _dense_body(x_ref, wq_ref, bq_ref, wk_ref, bk_ref, wv_ref, bv_ref,
                ws_ref, bs_ref, w1_ref, b1_ref, w2_ref, b2_ref,
                hw1b_ref, idx_ref, w2row_ref, b2v_ref):
    f32 = jnp.float32
    x = x_ref[...]
    q = jnp.dot(x, wq_ref[...], preferred_element_type=f32) + bq_ref[...]
    k = jnp.dot(x, wk_ref[...], preferred_element_type=f32) + bk_ref[...]
    v = jnp.dot(x, wv_ref[...], preferred_element_type=f32) + bv_ref[...]

    s = lax.dot_general(q, k, (((1,), (1,)), ((), ())),
                        preferred_element_type=f32)
    s = s / jnp.sqrt(f32(_D))
    row = lax.broadcasted_iota(jnp.int32, (_N, _N), 0)
    col = lax.broadcasted_iota(jnp.int32, (_N, _N), 1)
    diag = row == col
    s = jnp.where(diag, f32(-1e30), s)
    m = jnp.max(s, axis=1, keepdims=True)
    e = jnp.exp(s - m)
    ev = lax.dot_general(e, v, (((1,), (0,)), ((), ())),
                         preferred_element_type=f32)
    h = ev / jnp.sum(e, axis=1, keepdims=True)
    h = h + (jnp.dot(x, ws_ref[...], preferred_element_type=f32) + bs_ref[...])

    hh2 = h * h
    sq_col = jnp.sum(hh2, axis=1, keepdims=True)
    ones_row = jnp.zeros((1, _D), f32) + 1.0
    sq_row = lax.dot_general(ones_row, hh2, (((1,), (1,)), ((), ())),
                             preferred_element_type=f32,
                             precision=lax.Precision.HIGHEST)
    hh = lax.dot_general(h, h, (((1,), (1,)), ((), ())),
                         preferred_element_type=f32)
    d2 = (sq_col + sq_row) - 2.0 * hh
    inf = f32(jnp.inf)
    d2 = jnp.where(diag, inf, d2)

    big = jnp.int32(1 << 30)
    for t in range(_K):
        mn = jnp.min(d2, axis=1, keepdims=True)
        cand = jnp.where(d2 == mn, col, big)
        am = jnp.min(cand, axis=1, keepdims=True)
        d2 = jnp.where(col == am, inf, d2)
        idx_ref[pl.ds(t * _N, _N)] = jnp.transpose(am, (1, 0))[0]
    idx_ref[pl.ds(_K * _N, _N)] = lax.broadcasted_iota(jnp.int32, (1, _N), 1)[0]

    hw1b_ref[...] = jnp.dot(h, w1_ref[...],
                            preferred_element_type=f32) + b1_ref[...]
    w2row_ref[...] = jnp.transpose(w2_ref[...], (1, 0))[0]
    b2v_ref[...] = jnp.broadcast_to(b2_ref[...], (16,))


_dense_call = pl.pallas_call(
    _dense_body,
    out_shape=(
        jax.ShapeDtypeStruct((_N, _D), jnp.float32),
        jax.ShapeDtypeStruct((_F * _N,), jnp.int32),
        jax.ShapeDtypeStruct((_D,), jnp.float32),
        jax.ShapeDtypeStruct((16,), jnp.float32),
    ),
)


def _sc_body(hw1b_hbm, idx_hbm, w2_hbm, b2_hbm, out_hbm,
             idxv, rows, w2v, b2v, zloc, zall, outv, zsh, sem):
    w = lax.axis_index("s")
    base = w * _NPW

    for t in range(_F):
        pltpu.sync_copy(idx_hbm.at[pl.ds(t * _N + base, _NPW)],
                        idxv.at[pl.ds(t * _NPW, _NPW)])
    pltpu.sync_copy(w2_hbm, w2v)
    pltpu.sync_copy(b2_hbm, b2v)
    pltpu.async_copy(hw1b_hbm.at[idxv], rows, sem).wait()

    lane = jnp.arange(16, dtype=jnp.int32)

    for g in range(_NG):
        def body(j, zacc, _g=g):
            n = _g * 16 + j
            dot = jnp.zeros((16,), jnp.float32)
            for c in range(_D // 16):
                acc = rows[n, pl.ds(c * 16, 16)]
                for t in range(1, _F):
                    acc = acc + rows[t * _NPW + n, pl.ds(c * 16, 16)]
                gch = jnp.maximum(acc * 0.125, 0.0)
                dot = dot + gch * w2v[pl.ds(c * 16, 16)]
            zn = jnp.sum(dot)
            return jnp.where(lane == j, zn, zacc)

        z16 = lax.fori_loop(0, 16, body, jnp.zeros((16,), jnp.float32))
        zloc[pl.ds(g * 16, 16)] = z16 + b2v[...]

    pltpu.sync_copy(zloc, zsh.at[pl.ds(base, _NPW)])
    plsc.subcore_barrier()
    pltpu.sync_copy(zsh, zall)

    for g in range(_NG):
        acc = jnp.zeros((16,), jnp.float32)
        for t in range(_F):
            nbr = idxv[pl.ds(t * _NPW + g * 16, 16)]
            acc = acc + plsc.load_gather(zall, [nbr])
        outv[pl.ds(g * 16, 16)] = acc * 0.125
    pltpu.sync_copy(outv, out_hbm.at[pl.ds(base, _NPW)])


@functools.cache
def _sc_call():
  return pl.kernel(
    _sc_body,
    out_type=jax.ShapeDtypeStruct((_N,), jnp.float32),
    mesh=plsc.VectorSubcoreMesh(core_axis_name="c", subcore_axis_name="s",
                                num_cores=1, num_subcores=_NW),
    compiler_params=pltpu.CompilerParams(needs_layout_passes=False),
    scratch_types=[
        pltpu.VMEM((_NPW * _F,), jnp.int32),
        pltpu.VMEM((_NPW * _F, _D), jnp.float32),
        pltpu.VMEM((_D,), jnp.float32),
        pltpu.VMEM((16,), jnp.float32),
        pltpu.VMEM((_NPW,), jnp.float32),
        pltpu.VMEM((_N,), jnp.float32),
        pltpu.VMEM((_NPW,), jnp.float32),
        pltpu.VMEM_SHARED((_N,), jnp.float32),
        pltpu.SemaphoreType.DMA,
    ],
  )


def kernel(x, Wq, bq, Wk, bk, Wv, bv, Wskip, bskip, W1, b1, W2, b2):
    hw1b, idx_flat, w2row, b2v = _dense_call(
        x, Wq, bq, Wk, bk, Wv, bv, Wskip, bskip, W1, b1, W2, b2)
    out = _sc_call()(hw1b, idx_flat, w2row, b2v)
    return out.reshape(_N, 1)


# speedup vs baseline: 1.0965x; 1.0965x over previous
import functools

import jax
import jax.numpy as jnp
from jax import lax
from jax.experimental import pallas as pl
from jax.experimental.pallas import tpu as pltpu
from jax.experimental.pallas import tpu_sc as plsc

_N = 768
_D = 128
_K = 7
_F = _K + 1
_NW = 16
_NPW = _N // _NW
_NG = _NPW // 16


def _dense_body(x_ref, wq_ref, bq_ref, wk_ref, bk_ref, wv_ref, bv_ref,
                ws_ref, bs_ref, w1_ref, b1_ref, w2_ref, b2_ref,
                hw1b_ref, idx_ref, w2row_ref, b2v_ref):
    f32 = jnp.float32
    x = x_ref[...]
    q = jnp.dot(x, wq_ref[...], preferred_element_type=f32) + bq_ref[...]
    k = jnp.dot(x, wk_ref[...], preferred_element_type=f32) + bk_ref[...]
    v = jnp.dot(x, wv_ref[...], preferred_element_type=f32) + bv_ref[...]

    s = lax.dot_general(q, k, (((1,), (1,)), ((), ())),
                        preferred_element_type=f32)
    s = s / jnp.sqrt(f32(_D))
    row = lax.broadcasted_iota(jnp.int32, (_N, _N), 0)
    col = lax.broadcasted_iota(jnp.int32, (_N, _N), 1)
    diag = row == col
    s = jnp.where(diag, f32(-1e30), s)
    m = jnp.max(s, axis=1, keepdims=True)
    e = jnp.exp(s - m)
    ev = lax.dot_general(e, v, (((1,), (0,)), ((), ())),
                         preferred_element_type=f32)
    h = ev / jnp.sum(e, axis=1, keepdims=True)
    h = h + (jnp.dot(x, ws_ref[...], preferred_element_type=f32) + bs_ref[...])

    hh2 = h * h
    sq_col = jnp.sum(hh2, axis=1, keepdims=True)
    sq_row = jnp.transpose(sq_col, (1, 0))
    hh = lax.dot_general(h, h, (((1,), (1,)), ((), ())),
                         preferred_element_type=f32)
    d2 = (sq_col + sq_row) - 2.0 * hh
    inf = f32(jnp.inf)
    d2 = jnp.where(diag, inf, d2)

    big = jnp.int32(1 << 30)
    for t in range(_K):
        mn = jnp.min(d2, axis=1, keepdims=True)
        cand = jnp.where(d2 == mn, col, big)
        am = jnp.min(cand, axis=1, keepdims=True)
        d2 = jnp.where(col == am, inf, d2)
        idx_ref[pl.ds(t * _N, _N)] = jnp.transpose(am, (1, 0))[0]
    idx_ref[pl.ds(_K * _N, _N)] = lax.broadcasted_iota(jnp.int32, (1, _N), 1)[0]

    hw1b_ref[...] = jnp.dot(h, w1_ref[...],
                            preferred_element_type=f32) + b1_ref[...]
    w2row_ref[...] = jnp.transpose(w2_ref[...], (1, 0))[0]
    b2v_ref[...] = jnp.broadcast_to(b2_ref[...], (16,))


_dense_call = pl.pallas_call(
    _dense_body,
    out_shape=(
        jax.ShapeDtypeStruct((_N, _D), jnp.float32),
        jax.ShapeDtypeStruct((_F * _N,), jnp.int32),
        jax.ShapeDtypeStruct((_D,), jnp.float32),
        jax.ShapeDtypeStruct((16,), jnp.float32),
    ),
)


def _sc_body(hw1b_hbm, idx_hbm, w2_hbm, b2_hbm, out_hbm,
             idxv, rows, w2v, b2v, zloc, zall, outv, zsh, sem):
    w = lax.axis_index("s")
    base = w * _NPW

    handles = [
        pltpu.async_copy(idx_hbm.at[pl.ds(t * _N + base, _NPW)],
                         idxv.at[pl.ds(t * _NPW, _NPW)], sem)
        for t in range(_F)
    ]
    handles.append(pltpu.async_copy(w2_hbm, w2v, sem))
    handles.append(pltpu.async_copy(b2_hbm, b2v, sem))
    for h in handles:
        h.wait()
    pltpu.async_copy(hw1b_hbm.at[idxv], rows, sem).wait()

    lane = jnp.arange(16, dtype=jnp.int32)

    for g in range(_NG):
        def body(j, zacc, _g=g):
            n = _g * 16 + j
            dot = jnp.zeros((16,), jnp.float32)
            for c in range(_D // 16):
                acc = rows[n, pl.ds(c * 16, 16)]
                for t in range(1, _F):
                    acc = acc + rows[t * _NPW + n, pl.ds(c * 16, 16)]
                gch = jnp.maximum(acc * 0.125, 0.0)
                dot = dot + gch * w2v[pl.ds(c * 16, 16)]
            zn = jnp.sum(dot)
            return jnp.where(lane == j, zn, zacc)

        z16 = lax.fori_loop(0, 16, body, jnp.zeros((16,), jnp.float32))
        zloc[pl.ds(g * 16, 16)] = z16 + b2v[...]

    pltpu.sync_copy(zloc, zsh.at[pl.ds(base, _NPW)])
    plsc.subcore_barrier()
    pltpu.sync_copy(zsh, zall)

    for g in range(_NG):
        acc = jnp.zeros((16,), jnp.float32)
        for t in range(_F):
            nbr = idxv[pl.ds(t * _NPW + g * 16, 16)]
            acc = acc + plsc.load_gather(zall, [nbr])
        outv[pl.ds(g * 16, 16)] = acc * 0.125
    pltpu.sync_copy(outv, out_hbm.at[pl.ds(base, _NPW)])


@functools.cache
def _sc_call():
  return pl.kernel(
    _sc_body,
    out_type=jax.ShapeDtypeStruct((_N,), jnp.float32),
    mesh=plsc.VectorSubcoreMesh(core_axis_name="c", subcore_axis_name="s",
                                num_cores=1, num_subcores=_NW),
    compiler_params=pltpu.CompilerParams(needs_layout_passes=False),
    scratch_types=[
        pltpu.VMEM((_NPW * _F,), jnp.int32),
        pltpu.VMEM((_NPW * _F, _D), jnp.float32),
        pltpu.VMEM((_D,), jnp.float32),
        pltpu.VMEM((16,), jnp.float32),
        pltpu.VMEM((_NPW,), jnp.float32),
        pltpu.VMEM((_N,), jnp.float32),
        pltpu.VMEM((_NPW,), jnp.float32),
        pltpu.VMEM_SHARED((_N,), jnp.float32),
        pltpu.SemaphoreType.DMA,
    ],
  )


def kernel(x, Wq, bq, Wk, bk, Wv, bv, Wskip, bskip, W1, b1, W2, b2):
    hw1b, idx_flat, w2row, b2v = _dense_call(
        x, Wq, bq, Wk, bk, Wv, bv, Wskip, bskip, W1, b1, W2, b2)
    out = _sc_call()(hw1b, idx_flat, w2row, b2v)
    return out.reshape(_N, 1)
